# baseline (speedup 1.0000x reference)
import jax
import jax.numpy as jnp
from jax import lax
from jax.experimental import pallas as pl
from jax.experimental.pallas import tpu as pltpu
from jax.experimental.pallas import tpu_sc as plsc

N = 10000
D = 128
DH = D // 2
NLAYERS = 4
E_TOT = 320000
NC, NS = 2, 16
CHUNK = 128
ROWS = 158
EPAD = NS * ROWS * CHUNK
HROWS = ROWS // 2
NP = 10240
RPT = NP // NS
EPS = 1e-5

_MESH = plsc.VectorSubcoreMesh(
    core_axis_name="c", subcore_axis_name="s", num_cores=NC, num_subcores=NS
)


def _sc_deg_body(dst_hbm, out_hbm, dst_v, val_v, acc, sem):
    cid = lax.axis_index("c")
    sid = lax.axis_index("s")
    pltpu.async_copy(dst_hbm.at[sid], dst_v, sem)

    zero16 = jnp.zeros((16,), jnp.float32)

    def _zrow(r, carry):
        val_v[r, pl.ds(0, 16)] = zero16
        return carry

    lax.fori_loop(0, CHUNK, _zrow, None)
    base = sid * RPT
    for k in range(RPT // CHUNK):
        pltpu.sync_copy(val_v, acc.at[pl.ds(base + k * CHUNK, CHUNK)])

    one16 = jnp.ones((16,), jnp.float32)

    def _orow(r, carry):
        val_v[r, pl.ds(0, 16)] = one16
        return carry

    lax.fori_loop(0, CHUNK, _orow, None)
    pltpu.make_async_copy(dst_hbm.at[sid], dst_v, sem).wait()
    plsc.subcore_barrier()

    e0 = cid * HROWS

    def _step(e, carry):
        pltpu.sync_copy(val_v, acc.at[dst_v.at[e0 + e]], add=True)
        return carry

    lax.fori_loop(0, HROWS, _step, None)
    plsc.subcore_barrier()
    pltpu.sync_copy(acc.at[pl.ds(base, RPT)], out_hbm.at[cid, pl.ds(base, RPT)])


_sc_deg = pl.kernel(
    _sc_deg_body,
    out_type=jax.ShapeDtypeStruct((NC, NP, 16), jnp.float32),
    mesh=_MESH,
    compiler_params=pltpu.CompilerParams(use_tc_tiling_on_sc=False),
    scratch_types=[
        pltpu.VMEM((ROWS, CHUNK), jnp.int32),
        pltpu.VMEM((CHUNK, 16), jnp.float32),
        pltpu.VMEM_SHARED((NP, 16), jnp.float32),
        pltpu.SemaphoreType.DMA,
    ],
)


def _sc_msg_body(z_hbm, src_hbm, dst_hbm, out_hbm, src_v, dst_v,
                 b0, b1, g0, g1, acc):
    bufs = (b0, b1)
    gsems = (g0, g1)
    cid = lax.axis_index("c")
    sid = lax.axis_index("s")
    zv = z_hbm.at[cid]
    pltpu.async_copy(src_hbm.at[sid], src_v, gsems[0])
    pltpu.async_copy(dst_hbm.at[sid], dst_v, gsems[1])

    zero16 = jnp.zeros((16,), jnp.float32)

    def _zrow(r, carry):
        for c in range(DH // 16):
            bufs[0][r, pl.ds(c * 16, 16)] = zero16
        return carry

    lax.fori_loop(0, CHUNK, _zrow, None)
    base = sid * RPT
    for k in range(RPT // CHUNK):
        pltpu.sync_copy(bufs[0], acc.at[pl.ds(base + k * CHUNK, CHUNK)])
    pltpu.make_async_copy(src_hbm.at[sid], src_v, gsems[0]).wait()
    pltpu.make_async_copy(dst_hbm.at[sid], dst_v, gsems[1]).wait()
    plsc.subcore_barrier()

    pltpu.async_copy(zv.at[src_v.at[0]], bufs[0], gsems[0])
    pltpu.async_copy(zv.at[src_v.at[1]], bufs[1], gsems[1])

    def _step(i, carry):
        for b in range(2):
            e = i * 2 + b
            pltpu.make_async_copy(zv.at[src_v.at[e]], bufs[b], gsems[b]).wait()
            pltpu.sync_copy(bufs[b], acc.at[dst_v.at[e]], add=True)

            @pl.when(i < ROWS // 2 - 1)
            def _prefetch():
                pltpu.async_copy(zv.at[src_v.at[e + 2]], bufs[b], gsems[b])

        return carry

    lax.fori_loop(0, ROWS // 2, _step, None)
    plsc.subcore_barrier()
    pltpu.sync_copy(acc.at[pl.ds(base, RPT)], out_hbm.at[cid, pl.ds(base, RPT)])


_sc_msg = pl.kernel(
    _sc_msg_body,
    out_type=jax.ShapeDtypeStruct((NC, NP, DH), jnp.float32),
    mesh=_MESH,
    compiler_params=pltpu.CompilerParams(use_tc_tiling_on_sc=False),
    scratch_types=[
        pltpu.VMEM((ROWS, CHUNK), jnp.int32),
        pltpu.VMEM((ROWS, CHUNK), jnp.int32),
        pltpu.VMEM((CHUNK, DH), jnp.float32),
        pltpu.VMEM((CHUNK, DH), jnp.float32),
        pltpu.SemaphoreType.DMA,
        pltpu.SemaphoreType.DMA,
        pltpu.VMEM_SHARED((NP, DH), jnp.float32),
    ],
)


def _graph_norm_dense(h, w, b, ms):
    m = jnp.mean(h, axis=0, keepdims=True)
    hc = h - m * ms[None, :]
    var = jnp.mean(hc * hc, axis=0, keepdims=True)
    return w[None, :] * hc * lax.rsqrt(var + EPS) + b[None, :]


def _store_z_halves(z_ref, z):
    z_ref[0, 0:N, :] = z[:, 0:DH]
    z_ref[1, 0:N, :] = z[:, DH:D]
    pad = jnp.zeros((NP - N, DH), jnp.float32)
    z_ref[0, N:NP, :] = pad
    z_ref[1, N:NP, :] = pad


def _tc_init_body(x_ref, degp_ref, gnw_ref, gnb_ref, gnms_ref, w_ref,
                  z_ref, dinv_ref):
    dsum = degp_ref[0] + degp_ref[1]
    deg = jnp.sum(dsum, axis=1, keepdims=True) * (1.0 / 16.0) + 1.0
    dinv = lax.rsqrt(deg)
    dinv_ref[...] = dinv
    y = _graph_norm_dense(x_ref[...], gnw_ref[...], gnb_ref[...], gnms_ref[...])
    z = jnp.dot(y, w_ref[...], preferred_element_type=jnp.float32)
    _store_z_halves(z_ref, z * dinv[0:N])


def _tc_mid_body(p_ref, zp_ref, dinv_ref, b_ref, gnw_ref, gnb_ref, gnms_ref,
                 w_ref, z_ref):
    dinv = dinv_ref[0:N]
    agg = jnp.concatenate([p_ref[0, 0:N, :] + zp_ref[0, 0:N, :],
                           p_ref[1, 0:N, :] + zp_ref[1, 0:N, :]], axis=1)
    h = agg * dinv + b_ref[...][None, :]
    h = jnp.where(h >= 0.0, h, 0.01 * h)
    y = _graph_norm_dense(h, gnw_ref[...], gnb_ref[...], gnms_ref[...])
    z = jnp.dot(y, w_ref[...], preferred_element_type=jnp.float32)
    _store_z_halves(z_ref, z * dinv)


def _tc_final_body(p_ref, zp_ref, dinv_ref, b_ref, out_ref):
    dinv = dinv_ref[0:N]
    agg = jnp.concatenate([p_ref[0, 0:N, :] + zp_ref[0, 0:N, :],
                           p_ref[1, 0:N, :] + zp_ref[1, 0:N, :]], axis=1)
    out_ref[...] = agg * dinv + b_ref[...][None, :]


_tc_init = pl.pallas_call(
    _tc_init_body,
    out_shape=(
        jax.ShapeDtypeStruct((NC, NP, DH), jnp.float32),
        jax.ShapeDtypeStruct((NP, 1), jnp.float32),
    ),
)

_tc_mid = pl.pallas_call(
    _tc_mid_body,
    out_shape=jax.ShapeDtypeStruct((NC, NP, DH), jnp.float32),
)

_tc_final = pl.pallas_call(
    _tc_final_body,
    out_shape=jax.ShapeDtypeStruct((N, D), jnp.float32),
)


def kernel(x, edge_index, gn_w, gn_b, gn_ms, Ws, bs):
    ei = edge_index.astype(jnp.int32)
    pad = jnp.full((EPAD - E_TOT,), N, jnp.int32)
    srcp = jnp.concatenate([ei[0], pad]).reshape(NS, ROWS, CHUNK)
    dstp = jnp.concatenate([ei[1], pad]).reshape(NS, ROWS, CHUNK)

    degp = _sc_deg(dstp)
    z, dinv = _tc_init(x, degp, gn_w[0], gn_b[0], gn_ms[0], Ws[0])
    for i in range(1, NLAYERS):
        p = _sc_msg(z, srcp, dstp)
        z = _tc_mid(p, z, dinv, bs[i - 1], gn_w[i], gn_b[i], gn_ms[i], Ws[i])
    p = _sc_msg(z, srcp, dstp)
    return _tc_final(p, z, dinv, bs[NLAYERS - 1])


# speedup vs baseline: 1.1670x; 1.1670x over previous
import jax
import jax.numpy as jnp
from jax import lax
from jax.experimental import pallas as pl
from jax.experimental.pallas import tpu as pltpu
from jax.experimental.pallas import tpu_sc as plsc

N = 10000
D = 128
DH = D // 2
NLAYERS = 4
E_TOT = 320000 + N
NC, NS = 2, 16
CHUNK = 128
ROWS = 162
EPAD = NS * ROWS * CHUNK
HROWS = ROWS // 2
NP = 10240
RPT = NP // NS
EPS = 1e-5

_MESH = plsc.VectorSubcoreMesh(
    core_axis_name="c", subcore_axis_name="s", num_cores=NC, num_subcores=NS
)


def _sc_deg_body(dst_hbm, out_hbm, dst_v, val_v, acc, sem):
    cid = lax.axis_index("c")
    sid = lax.axis_index("s")
    pltpu.async_copy(dst_hbm.at[sid], dst_v, sem)

    zero16 = jnp.zeros((16,), jnp.float32)

    def _zrow(r, carry):
        val_v[r, pl.ds(0, 16)] = zero16
        return carry

    lax.fori_loop(0, CHUNK, _zrow, None)
    base = sid * RPT
    for k in range(RPT // CHUNK):
        pltpu.sync_copy(val_v, acc.at[pl.ds(base + k * CHUNK, CHUNK)])

    one16 = jnp.ones((16,), jnp.float32)

    def _orow(r, carry):
        val_v[r, pl.ds(0, 16)] = one16
        return carry

    lax.fori_loop(0, CHUNK, _orow, None)
    pltpu.make_async_copy(dst_hbm.at[sid], dst_v, sem).wait()
    plsc.subcore_barrier()

    e0 = cid * HROWS

    def _step(e, carry):
        pltpu.sync_copy(val_v, acc.at[dst_v.at[e0 + e]], add=True)
        return carry

    lax.fori_loop(0, HROWS, _step, None)
    plsc.subcore_barrier()
    pltpu.sync_copy(acc.at[pl.ds(base, RPT)], out_hbm.at[cid, pl.ds(base, RPT)])


_sc_deg = pl.kernel(
    _sc_deg_body,
    out_type=jax.ShapeDtypeStruct((NC, NP, 16), jnp.float32),
    mesh=_MESH,
    compiler_params=pltpu.CompilerParams(use_tc_tiling_on_sc=False),
    scratch_types=[
        pltpu.VMEM((ROWS, CHUNK), jnp.int32),
        pltpu.VMEM((CHUNK, 16), jnp.float32),
        pltpu.VMEM_SHARED((NP, 16), jnp.float32),
        pltpu.SemaphoreType.DMA,
    ],
)


def _sc_msg_body(z_hbm, src_hbm, dst_hbm, out_hbm, src_v, dst_v,
                 b0, b1, g0, g1, acc):
    bufs = (b0, b1)
    gsems = (g0, g1)
    cid = lax.axis_index("c")
    sid = lax.axis_index("s")
    zv = z_hbm.at[cid]
    pltpu.async_copy(src_hbm.at[sid], src_v, gsems[0])
    pltpu.async_copy(dst_hbm.at[sid], dst_v, gsems[1])

    zero16 = jnp.zeros((16,), jnp.float32)

    def _zrow(r, carry):
        for c in range(DH // 16):
            bufs[0][r, pl.ds(c * 16, 16)] = zero16
        return carry

    lax.fori_loop(0, CHUNK, _zrow, None)
    base = sid * RPT
    for k in range(RPT // CHUNK):
        pltpu.sync_copy(bufs[0], acc.at[pl.ds(base + k * CHUNK, CHUNK)])
    pltpu.make_async_copy(src_hbm.at[sid], src_v, gsems[0]).wait()
    pltpu.make_async_copy(dst_hbm.at[sid], dst_v, gsems[1]).wait()
    plsc.subcore_barrier()

    pltpu.async_copy(zv.at[src_v.at[0]], bufs[0], gsems[0])
    pltpu.async_copy(zv.at[src_v.at[1]], bufs[1], gsems[1])

    def _step(i, carry):
        for b in range(2):
            e = i * 2 + b
            pltpu.make_async_copy(zv.at[src_v.at[e]], bufs[b], gsems[b]).wait()
            pltpu.sync_copy(bufs[b], acc.at[dst_v.at[e]], add=True)

            @pl.when(i < ROWS // 2 - 1)
            def _prefetch():
                pltpu.async_copy(zv.at[src_v.at[e + 2]], bufs[b], gsems[b])

        return carry

    lax.fori_loop(0, ROWS // 2, _step, None)
    plsc.subcore_barrier()
    for k in range(RPT // CHUNK):
        r0 = base + k * CHUNK
        pltpu.sync_copy(acc.at[pl.ds(r0, CHUNK)], out_hbm.at[cid, pl.ds(r0, CHUNK)])


_sc_msg = pl.kernel(
    _sc_msg_body,
    out_type=jax.ShapeDtypeStruct((NC, NP, DH), jnp.float32),
    mesh=_MESH,
    compiler_params=pltpu.CompilerParams(use_tc_tiling_on_sc=False),
    scratch_types=[
        pltpu.VMEM((ROWS, CHUNK), jnp.int32),
        pltpu.VMEM((ROWS, CHUNK), jnp.int32),
        pltpu.VMEM((CHUNK, DH), jnp.float32),
        pltpu.VMEM((CHUNK, DH), jnp.float32),
        pltpu.SemaphoreType.DMA,
        pltpu.SemaphoreType.DMA,
        pltpu.VMEM_SHARED((NP, DH), jnp.float32),
    ],
)


def _graph_norm_dense(h, w, b, ms):
    m = jnp.mean(h, axis=0, keepdims=True)
    hc = h - m * ms[None, :]
    var = jnp.mean(hc * hc, axis=0, keepdims=True)
    return w[None, :] * hc * lax.rsqrt(var + EPS) + b[None, :]


def _store_z_halves(z_ref, z):
    z_ref[0, 0:N, :] = z[:, 0:DH]
    z_ref[1, 0:N, :] = z[:, DH:D]
    pad = jnp.zeros((NP - N, DH), jnp.float32)
    z_ref[0, N:NP, :] = pad
    z_ref[1, N:NP, :] = pad


def _tc_init_body(x_ref, degp_ref, gnw_ref, gnb_ref, gnms_ref, w_ref,
                  z_ref, dinv_ref):
    dsum = degp_ref[0] + degp_ref[1]
    deg = jnp.sum(dsum, axis=1, keepdims=True) * (1.0 / 16.0)
    dinv = jnp.where(deg > 0.0, lax.rsqrt(jnp.maximum(deg, 1.0)), 0.0)
    dinv_ref[...] = dinv
    y = _graph_norm_dense(x_ref[...], gnw_ref[...], gnb_ref[...], gnms_ref[...])
    z = jnp.dot(y, w_ref[...], preferred_element_type=jnp.float32)
    _store_z_halves(z_ref, z * dinv[0:N])


def _tc_mid_body(p_ref, dinv_ref, b_ref, gnw_ref, gnb_ref, gnms_ref, w_ref,
                 z_ref):
    dinv = dinv_ref[0:N]
    agg = jnp.concatenate([p_ref[0, 0:N, :], p_ref[1, 0:N, :]], axis=1)
    h = agg * dinv + b_ref[...][None, :]
    h = jnp.where(h >= 0.0, h, 0.01 * h)
    y = _graph_norm_dense(h, gnw_ref[...], gnb_ref[...], gnms_ref[...])
    z = jnp.dot(y, w_ref[...], preferred_element_type=jnp.float32)
    _store_z_halves(z_ref, z * dinv)


def _tc_final_body(p_ref, dinv_ref, b_ref, out_ref):
    dinv = dinv_ref[0:N]
    agg = jnp.concatenate([p_ref[0, 0:N, :], p_ref[1, 0:N, :]], axis=1)
    out_ref[...] = agg * dinv + b_ref[...][None, :]


_tc_init = pl.pallas_call(
    _tc_init_body,
    out_shape=(
        jax.ShapeDtypeStruct((NC, NP, DH), jnp.float32),
        jax.ShapeDtypeStruct((NP, 1), jnp.float32),
    ),
)

_tc_mid = pl.pallas_call(
    _tc_mid_body,
    out_shape=jax.ShapeDtypeStruct((NC, NP, DH), jnp.float32),
)

_tc_final = pl.pallas_call(
    _tc_final_body,
    out_shape=jax.ShapeDtypeStruct((N, D), jnp.float32),
)


def kernel(x, edge_index, gn_w, gn_b, gn_ms, Ws, bs):
    ei = edge_index.astype(jnp.int32)
    loops = jnp.arange(N, dtype=jnp.int32)
    pad = jnp.full((EPAD - E_TOT,), N, jnp.int32)
    srcp = jnp.concatenate([ei[0], loops, pad]).reshape(NS, ROWS, CHUNK)
    dstp = jnp.concatenate([ei[1], loops, pad]).reshape(NS, ROWS, CHUNK)

    degp = _sc_deg(dstp)
    z, dinv = _tc_init(x, degp, gn_w[0], gn_b[0], gn_ms[0], Ws[0])
    for i in range(1, NLAYERS):
        p = _sc_msg(z, srcp, dstp)
        z = _tc_mid(p, dinv, bs[i - 1], gn_w[i], gn_b[i], gn_ms[i], Ws[i])
    p = _sc_msg(z, srcp, dstp)
    return _tc_final(p, dinv, bs[NLAYERS - 1])
